# baseline (speedup 1.0000x reference)
import functools

import jax
import jax.numpy as jnp
from jax import lax
from jax.experimental import pallas as pl
from jax.experimental.pallas import tpu as pltpu
from jax.experimental.pallas import tpu_sc as plsc

N = 100000
B = 10000
K = 16
D = 256
A = 64

BB = 128
GRID = (B + BB - 1) // BB
BP = 10240
CP = 12288
NBP = K * BP

NW = 32
CPW = CP // NW
NPW = NBP // NW
CH_C = 128
CH_N = 160
NCH_C = CPW // CH_C
NCH_N = NPW // CH_N


def _sc_gather(table_f32, table_u32, idx_c, idx_n):
    mesh = plsc.VectorSubcoreMesh(core_axis_name="c", subcore_axis_name="s")

    @functools.partial(
        pl.kernel,
        mesh=mesh,
        out_type=(
            jax.ShapeDtypeStruct((CP, D), jnp.float32),
            jax.ShapeDtypeStruct((NBP, D // 2), jnp.int32),
        ),
        scratch_types=[
            pltpu.VMEM((CH_C,), jnp.int32),
            pltpu.VMEM((CH_N,), jnp.int32),
            pltpu.VMEM((CH_N,), jnp.int32),
            pltpu.VMEM((CH_C, D), jnp.float32),
            pltpu.VMEM((CH_C, D), jnp.float32),
            pltpu.VMEM((CH_N, D // 2), jnp.int32),
            pltpu.VMEM((CH_N, D // 2), jnp.int32),
            pltpu.SemaphoreType.DMA,
            pltpu.SemaphoreType.DMA,
            pltpu.SemaphoreType.DMA,
            pltpu.SemaphoreType.DMA,
            pltpu.SemaphoreType.DMA,
        ],
    )
    def gather_kernel(tf_hbm, tu_hbm, ixc_hbm, ixn_hbm, outc_hbm, outn_hbm,
                      ibc, ibn0, ibn1, cb0, cb1, nb0, nb1,
                      sic, sin0, sin1, sg0, sg1):
        nc = 2
        wid = lax.axis_index("s") * nc + lax.axis_index("c")
        cbase = wid * CPW
        nbase = wid * NPW

        cbufs = (cb0, cb1)
        sgs = (sg0, sg1)

        def cg_start(c, b):
            pltpu.sync_copy(ixc_hbm.at[pl.ds(cbase + c * CH_C, CH_C)], ibc)
            pltpu.async_copy(tf_hbm.at[ibc], cbufs[b], sgs[b])

        def cg_finish(c, b):
            pltpu.make_async_copy(
                tf_hbm.at[ibc], cbufs[b], sgs[b]).wait()
            pltpu.sync_copy(cbufs[b],
                            outc_hbm.at[pl.ds(cbase + c * CH_C, CH_C)])

        for c in range(NCH_C):
            b = c % 2
            cg_start(c, b)
            cg_finish(c, b)

        nbufs = (nb0, nb1)
        ibns = (ibn0, ibn1)
        sins = (sin0, sin1)

        def ng_ix(c, b):
            pltpu.async_copy(
                ixn_hbm.at[pl.ds(nbase + c * CH_N, CH_N)], ibns[b], sins[b])

        def ng_start(c, b):
            pltpu.make_async_copy(
                ixn_hbm.at[pl.ds(nbase, CH_N)], ibns[b], sins[b]).wait()
            pltpu.async_copy(tu_hbm.at[ibns[b]], nbufs[b], sgs[b])

        def ng_finish(c, b):
            pltpu.make_async_copy(
                tu_hbm.at[ibns[b]], nbufs[b], sgs[b]).wait()
            pltpu.sync_copy(nbufs[b],
                            outn_hbm.at[pl.ds(nbase + c * CH_N, CH_N)])

        ng_ix(0, 0)
        ng_ix(1, 1)
        ng_start(0, 0)

        def body(i, carry):
            c0 = i * 2
            ng_start(c0 + 1, 1)
            ng_finish(c0, 0)

            @pl.when(c0 + 2 < NCH_N)
            def _():
                ng_ix(c0 + 2, 0)
                ng_start(c0 + 2, 0)

            ng_finish(c0 + 1, 1)

            @pl.when(c0 + 3 < NCH_N)
            def _():
                ng_ix(c0 + 3, 1)
            return carry

        lax.fori_loop(0, NCH_N // 2, body, 0)

    return gather_kernel(table_f32, table_u32, idx_c, idx_n)


def _tc_body(*refs):
    cen_ref = refs[0]
    nb_refs = refs[1:1 + K]
    (nbw_ref, wq_ref, wk_ref, wg_ref, bg_ref, wo1_ref, wo2_ref, bo_ref,
     gamma_ref, beta_ref, out_ref) = refs[1 + K:]
    scale = A ** -0.5
    himask = jnp.int32(-65536)
    cen = cen_ref[...]
    q = jnp.dot(cen, wq_ref[...], preferred_element_type=jnp.float32)
    wk = wk_ref[...]
    nbs = []
    for k in range(K):
        x = nb_refs[k][...]
        lo = lax.bitcast_convert_type(x << 16, jnp.float32)
        hi = lax.bitcast_convert_type(x & himask, jnp.float32)
        nbs.append(jnp.concatenate([lo, hi], axis=1))
    cols = []
    for k in range(K):
        kp = jnp.dot(nbs[k], wk, preferred_element_type=jnp.float32)
        cols.append(jnp.sum(q * kp, axis=1, keepdims=True))
    s = jnp.concatenate(cols, axis=1) * scale
    s = s + jnp.log(jnp.maximum(nbw_ref[...], 1e-6))
    s = s - jnp.max(s, axis=1, keepdims=True)
    e = jnp.exp(s)
    attn = e / jnp.sum(e, axis=1, keepdims=True)
    ctx = attn[:, 0:1] * nbs[0]
    for k in range(1, K):
        ctx = ctx + attn[:, k:k + 1] * nbs[k]
    gate = jax.nn.sigmoid(
        jnp.dot(cen, wg_ref[...], preferred_element_type=jnp.float32)
        + bg_ref[...])
    ctx = gate * ctx
    enr = (jnp.dot(cen, wo1_ref[...], preferred_element_type=jnp.float32)
           + jnp.dot(ctx, wo2_ref[...], preferred_element_type=jnp.float32)
           + bo_ref[...])
    x = enr + cen
    mean = jnp.mean(x, axis=1, keepdims=True)
    xc = x - mean
    var = jnp.mean(xc * xc, axis=1, keepdims=True)
    out_ref[...] = gamma_ref[...] * xc * lax.rsqrt(var + 1e-5) + beta_ref[...]


def _nb_spec(k):
    off = k * (BP // BB)
    return pl.BlockSpec((BB, D // 2), lambda b, off=off: (b + off, 0))


def _tc_compute(gc, gn, nbw, wq, wk, wg, bg, wo1, wo2, bo, gamma, beta):
    return pl.pallas_call(
        _tc_body,
        grid=(GRID,),
        in_specs=[pl.BlockSpec((BB, D), lambda b: (b, 0))]
        + [_nb_spec(k) for k in range(K)]
        + [
            pl.BlockSpec((BB, K), lambda b: (b, 0)),
            pl.BlockSpec((D, A), lambda b: (0, 0)),
            pl.BlockSpec((D, A), lambda b: (0, 0)),
            pl.BlockSpec((D, D), lambda b: (0, 0)),
            pl.BlockSpec((1, D), lambda b: (0, 0)),
            pl.BlockSpec((D, D), lambda b: (0, 0)),
            pl.BlockSpec((D, D), lambda b: (0, 0)),
            pl.BlockSpec((1, D), lambda b: (0, 0)),
            pl.BlockSpec((1, D), lambda b: (0, 0)),
            pl.BlockSpec((1, D), lambda b: (0, 0)),
        ],
        out_specs=pl.BlockSpec((BB, D), lambda b: (b, 0)),
        out_shape=jax.ShapeDtypeStruct((GRID * BB, D), jnp.float32),
    )(gc, *([gn] * K), nbw, wq, wk, wg, bg, wo1, wo2, bo, gamma, beta)


def kernel(all_embs, center_idx, nb_idx, nb_weights, Wq, Wk, Wg, bg, Wo, bo,
           gamma, beta):
    t16 = all_embs.astype(jnp.bfloat16).reshape(N, D // 2, 2)
    tu32 = lax.bitcast_convert_type(t16, jnp.int32)
    ci = jnp.pad(center_idx.astype(jnp.int32), (0, CP - B))
    nbt = jnp.pad(nb_idx.astype(jnp.int32).T, ((0, 0), (0, BP - B)))
    idx_n = nbt.reshape(-1)
    gc, gn = _sc_gather(all_embs, tu32, ci, idx_n)
    perm = jnp.concatenate([jnp.arange(0, D, 2), jnp.arange(1, D, 2)])
    nbw = jnp.pad(nb_weights, ((0, GRID * BB - B), (0, 0)), constant_values=1.0)
    out = _tc_compute(
        gc, gn, nbw, Wq, Wk[perm], Wg[:, perm], bg[perm].reshape(1, D),
        Wo[:D], Wo[D:][perm], bo.reshape(1, D), gamma.reshape(1, D),
        beta.reshape(1, D))
    return out[:B]


# speedup vs baseline: 1.2441x; 1.2441x over previous
import functools

import jax
import jax.numpy as jnp
from jax import lax
from jax.experimental import pallas as pl
from jax.experimental.pallas import tpu as pltpu
from jax.experimental.pallas import tpu_sc as plsc

N = 100000
B = 10000
K = 16
D = 256
A = 64

BB = 128
BP = 10240
PH = 2
PHC = BP // PH
PHG = PHC // BB

NW = 32
CHUNK = 128
PROWS = PHC * (K + 1)
NCH = -(-PROWS // (NW * CHUNK))
RPW = NCH * CHUNK
P = NW * RPW


def _sc_gather(table, idx_ph):
    mesh = plsc.VectorSubcoreMesh(core_axis_name="c", subcore_axis_name="s")

    @functools.partial(
        pl.kernel,
        mesh=mesh,
        out_type=jax.ShapeDtypeStruct((P, D), jnp.float32),
        scratch_types=[
            pltpu.VMEM((RPW,), jnp.int32),
            pltpu.VMEM((CHUNK, D), jnp.float32),
            pltpu.VMEM((CHUNK, D), jnp.float32),
            pltpu.SemaphoreType.DMA,
            pltpu.SemaphoreType.DMA,
        ],
    )
    def gather_kernel(table_hbm, idx_hbm, out_hbm, idx_v, buf0, buf1,
                      sg0, sg1):
        nc = 2
        wid = lax.axis_index("s") * nc + lax.axis_index("c")
        base = wid * RPW
        bufs = (buf0, buf1)
        sgs = (sg0, sg1)
        pltpu.sync_copy(idx_hbm.at[pl.ds(base, RPW)], idx_v)

        def g_start(c, b):
            pltpu.async_copy(
                table_hbm.at[idx_v.at[pl.ds(c * CHUNK, CHUNK)]],
                bufs[b], sgs[b])

        def finish(c, b):
            pltpu.make_async_copy(
                table_hbm.at[idx_v.at[pl.ds(0, CHUNK)]], bufs[b],
                sgs[b]).wait()
            pltpu.sync_copy(bufs[b],
                            out_hbm.at[pl.ds(base + c * CHUNK, CHUNK)])

        g_start(0, 0)

        def body(i, carry):
            c0 = i * 2
            g_start(c0 + 1, 1)
            finish(c0, 0)

            @pl.when(c0 + 2 < NCH)
            def _():
                g_start(c0 + 2, 0)

            finish(c0 + 1, 1)
            return carry

        lax.fori_loop(0, NCH // 2, body, 0)

    return gather_kernel(table, idx_ph)


def _tc_body(*refs):
    cen_ref = refs[0]
    nb_refs = refs[1:1 + K]
    (nbw_ref, wq_ref, wk_ref, wg_ref, bg_ref, wo_ref, bo_ref, gamma_ref,
     beta_ref, out_ref) = refs[1 + K:]
    scale = A ** -0.5
    cen = cen_ref[...]
    q = jnp.dot(cen, wq_ref[...], preferred_element_type=jnp.float32)
    wk = wk_ref[...]
    nbs = [r[...] for r in nb_refs]
    cols = []
    for k in range(K):
        kp = jnp.dot(nbs[k], wk, preferred_element_type=jnp.float32)
        cols.append(jnp.sum(q * kp, axis=1, keepdims=True))
    s = jnp.concatenate(cols, axis=1) * scale
    s = s + jnp.log(jnp.maximum(nbw_ref[...], 1e-6))
    s = s - jnp.max(s, axis=1, keepdims=True)
    e = jnp.exp(s)
    attn = e / jnp.sum(e, axis=1, keepdims=True)
    ctx = attn[:, 0:1] * nbs[0]
    for k in range(1, K):
        ctx = ctx + attn[:, k:k + 1] * nbs[k]
    gate = jax.nn.sigmoid(
        jnp.dot(cen, wg_ref[...], preferred_element_type=jnp.float32)
        + bg_ref[...])
    ctx = gate * ctx
    wo = wo_ref[...]
    enr = (jnp.dot(cen, wo[:D], preferred_element_type=jnp.float32)
           + jnp.dot(ctx, wo[D:], preferred_element_type=jnp.float32)
           + bo_ref[...])
    x = enr + cen
    mean = jnp.mean(x, axis=1, keepdims=True)
    xc = x - mean
    var = jnp.mean(xc * xc, axis=1, keepdims=True)
    out_ref[...] = gamma_ref[...] * xc * lax.rsqrt(var + 1e-5) + beta_ref[...]


def _nb_spec(k):
    off = (PHC + k * PHC) // BB
    return pl.BlockSpec((BB, D), lambda b, off=off: (b + off, 0))


def _tc_compute(g, nbw, wq, wk, wg, bg, wo, bo, gamma, beta):
    return pl.pallas_call(
        _tc_body,
        grid=(PHG,),
        in_specs=[pl.BlockSpec((BB, D), lambda b: (b, 0))]
        + [_nb_spec(k) for k in range(K)]
        + [
            pl.BlockSpec((BB, K), lambda b: (b, 0)),
            pl.BlockSpec((D, A), lambda b: (0, 0)),
            pl.BlockSpec((D, A), lambda b: (0, 0)),
            pl.BlockSpec((D, D), lambda b: (0, 0)),
            pl.BlockSpec((1, D), lambda b: (0, 0)),
            pl.BlockSpec((2 * D, D), lambda b: (0, 0)),
            pl.BlockSpec((1, D), lambda b: (0, 0)),
            pl.BlockSpec((1, D), lambda b: (0, 0)),
            pl.BlockSpec((1, D), lambda b: (0, 0)),
        ],
        out_specs=pl.BlockSpec((BB, D), lambda b: (b, 0)),
        out_shape=jax.ShapeDtypeStruct((PHC, D), jnp.float32),
    )(g, *([g] * K), nbw, wq, wk, wg, bg, wo, bo, gamma, beta)


def kernel(all_embs, center_idx, nb_idx, nb_weights, Wq, Wk, Wg, bg, Wo, bo,
           gamma, beta):
    ci = jnp.pad(center_idx.astype(jnp.int32), (0, BP - B))
    nbt = jnp.pad(nb_idx.astype(jnp.int32).T, ((0, 0), (0, BP - B)))
    nbw = jnp.pad(nb_weights, ((0, BP - B), (0, 0)), constant_values=1.0)
    bg2, bo2 = bg.reshape(1, D), bo.reshape(1, D)
    ga2, be2 = gamma.reshape(1, D), beta.reshape(1, D)
    outs = []
    for p in range(PH):
        lo = p * PHC
        idx_p = jnp.concatenate([
            lax.dynamic_slice_in_dim(ci, lo, PHC),
            lax.dynamic_slice_in_dim(nbt, lo, PHC, axis=1).reshape(-1),
            jnp.zeros((P - PROWS,), jnp.int32),
        ])
        g = _sc_gather(all_embs, idx_p)
        outs.append(_tc_compute(
            g, lax.dynamic_slice_in_dim(nbw, lo, PHC), Wq, Wk, Wg, bg2, Wo,
            bo2, ga2, be2))
    return jnp.concatenate(outs)[:B]
